# baseline (speedup 1.0000x reference)
import functools

import jax
import jax.numpy as jnp
from jax import lax
from jax.experimental import pallas as pl
from jax.experimental.pallas import tpu as pltpu
from jax.experimental.pallas import tpu_sc as plsc

_BATCH = 16384
_EDIM = 16
_NC = 2
_NS = 16
_NW = _NC * _NS
_BPW = _BATCH // _NW
_GRP = _BPW // 16


def _mf_body(user_ref, item_ref, ut_tbl, vt_tbl, out_ref,
             uidx, iidx, ublk, vblk, outv,
             sem0, sem1, sem2, sem3, sem4, sem5, sem6, sem7):
    wid = lax.axis_index("s") * _NC + lax.axis_index("c")
    row0 = pl.multiple_of(wid * _GRP, 8)
    pltpu.sync_copy(user_ref.at[pl.ds(row0, _GRP)], uidx)
    pltpu.sync_copy(item_ref.at[pl.ds(row0, _GRP)], iidx)

    sems = [sem0, sem1, sem2, sem3, sem4, sem5, sem6, sem7]
    lane = lax.iota(jnp.int32, 16)

    def fire_quarter(g, qq):
        uvec = uidx[g]
        vvec = iidx[g]
        for j in range(2):
            l = 2 * qq + j
            ru = uvec[l]
            rv = vvec[l]
            offu = lax.div(ru, 128) * 128
            offv = lax.div(rv, 128) * 128
            slot = 2 * qq + j
            pltpu.async_copy(ut_tbl.at[pl.ds(0, 8), pl.ds(offu, 128)],
                             ublk.at[slot, pl.ds(0, 8)], sems[qq])
            pltpu.async_copy(ut_tbl.at[pl.ds(8, 8), pl.ds(offu, 128)],
                             ublk.at[slot, pl.ds(8, 8)], sems[qq])
            pltpu.async_copy(vt_tbl.at[pl.ds(0, 8), pl.ds(offv, 128)],
                             vblk.at[slot, pl.ds(0, 8)], sems[qq])
            pltpu.async_copy(vt_tbl.at[pl.ds(8, 8), pl.ds(offv, 128)],
                             vblk.at[slot, pl.ds(8, 8)], sems[qq])

    def drain_quarter(qq):
        for _ in range(8):
            pltpu.make_async_copy(ut_tbl.at[pl.ds(0, 8), pl.ds(0, 128)],
                                  ublk.at[2 * qq, pl.ds(0, 8)], sems[qq]).wait()

    for qq in range(8):
        fire_quarter(0, qq)

    def group(g, _):
        uvec = uidx[g]
        vvec = iidx[g]
        grp = jnp.zeros((16,), jnp.float32)
        for qq in range(8):
            drain_quarter(qq)
            for j in range(2):
                l = 2 * qq + j
                ru = uvec[l]
                rv = vvec[l]
                cu = lax.rem(ru, 128)
                cv = lax.rem(rv, 128)
                segu = lax.div(cu, 16) * 16
                segv = lax.div(cv, 16) * 16
                su = jnp.full((16,), lax.rem(cu, 16), jnp.int32)
                sv = jnp.full((16,), lax.rem(cv, 16), jnp.int32)
                slot = 2 * qq + j
                acc = jnp.zeros((16,), jnp.float32)
                for d in range(_EDIM):
                    bu = jnp.take(ublk[slot, d, pl.ds(segu, 16)], su)
                    bv = jnp.take(vblk[slot, d, pl.ds(segv, 16)], sv)
                    acc = acc + bu * bv
                grp = jnp.where(lane == l, acc, grp)
            @pl.when(g < _GRP - 1)
            def _():
                fire_quarter(g + 1, qq)
        outv[g] = 1.0 / (1.0 + jnp.exp(-grp))
        return 0

    lax.fori_loop(0, _GRP, group, 0)
    pltpu.sync_copy(outv, out_ref.at[pl.ds(row0, _GRP)])


_mf_sc = functools.partial(
    pl.kernel,
    out_type=jax.ShapeDtypeStruct((_NW * _GRP, 16), jnp.float32),
    mesh=plsc.VectorSubcoreMesh(
        core_axis_name="c", subcore_axis_name="s",
        num_cores=_NC, num_subcores=_NS),
    scratch_types=[
        pltpu.VMEM((_GRP, 16), jnp.int32),
        pltpu.VMEM((_GRP, 16), jnp.int32),
        pltpu.VMEM((16, _EDIM, 128), jnp.float32),
        pltpu.VMEM((16, _EDIM, 128), jnp.float32),
        pltpu.VMEM((_GRP, 16), jnp.float32),
        pltpu.SemaphoreType.DMA,
        pltpu.SemaphoreType.DMA,
        pltpu.SemaphoreType.DMA,
        pltpu.SemaphoreType.DMA,
        pltpu.SemaphoreType.DMA,
        pltpu.SemaphoreType.DMA,
        pltpu.SemaphoreType.DMA,
        pltpu.SemaphoreType.DMA,
    ],
)(_mf_body)


def kernel(user, item, U, V):
    u2 = user.astype(jnp.int32).reshape(_NW * _GRP, 16)
    i2 = item.astype(jnp.int32).reshape(_NW * _GRP, 16)
    out = _mf_sc(u2, i2, U.T, V.T)
    return out.reshape(_BATCH)


# speedup vs baseline: 1.2064x; 1.2064x over previous
import functools

import jax
import jax.numpy as jnp
from jax import lax
from jax.experimental import pallas as pl
from jax.experimental.pallas import tpu as pltpu
from jax.experimental.pallas import tpu_sc as plsc

_BATCH = 16384
_EDIM = 16
_NC = 2
_NS = 16
_NW = _NC * _NS
_BPW = _BATCH // _NW
_GRP = _BPW // 16


def _mf_body(user_ref, item_ref, ut_tbl, vt_tbl, out_ref,
             uidx, iidx, ublk, vblk, outv,
             sem0, sem1, sem2, sem3, sem4, sem5, sem6, sem7):
    wid = lax.axis_index("s") * _NC + lax.axis_index("c")
    row0 = pl.multiple_of(wid * _GRP, 8)
    pltpu.sync_copy(user_ref.at[pl.ds(row0, _GRP)], uidx)
    pltpu.sync_copy(item_ref.at[pl.ds(row0, _GRP)], iidx)

    sems = [sem0, sem1, sem2, sem3, sem4, sem5, sem6, sem7]
    lane = lax.iota(jnp.int32, 16)

    def fire_quarter(g, qq):
        uvec = uidx[g]
        vvec = iidx[g]
        for j in range(2):
            l = 2 * qq + j
            ru = uvec[l]
            rv = vvec[l]
            offu = lax.div(ru, 128) * 128
            offv = lax.div(rv, 128) * 128
            slot = 2 * qq + j
            pltpu.async_copy(ut_tbl.at[:, pl.ds(offu, 128)], ublk.at[slot], sems[qq])
            pltpu.async_copy(vt_tbl.at[:, pl.ds(offv, 128)], vblk.at[slot], sems[qq])

    def drain_quarter(qq):
        for _ in range(4):
            pltpu.make_async_copy(ut_tbl.at[:, pl.ds(0, 128)],
                                  ublk.at[2 * qq], sems[qq]).wait()

    for qq in range(8):
        fire_quarter(0, qq)

    def group(g, _):
        uvec = uidx[g]
        vvec = iidx[g]
        grp = jnp.zeros((16,), jnp.float32)
        for qq in range(8):
            drain_quarter(qq)
            for j in range(2):
                l = 2 * qq + j
                ru = uvec[l]
                rv = vvec[l]
                cu = lax.rem(ru, 128)
                cv = lax.rem(rv, 128)
                segu = lax.div(cu, 16) * 16
                segv = lax.div(cv, 16) * 16
                su = jnp.full((16,), lax.rem(cu, 16), jnp.int32)
                sv = jnp.full((16,), lax.rem(cv, 16), jnp.int32)
                slot = 2 * qq + j
                acc = jnp.zeros((16,), jnp.float32)
                for d in range(_EDIM):
                    bu = jnp.take(ublk[slot, d, pl.ds(segu, 16)], su)
                    bv = jnp.take(vblk[slot, d, pl.ds(segv, 16)], sv)
                    acc = acc + bu * bv
                grp = jnp.where(lane == l, acc, grp)
            @pl.when(g < _GRP - 1)
            def _():
                fire_quarter(g + 1, qq)
        outv[g] = 1.0 / (1.0 + jnp.exp(-grp))
        return 0

    lax.fori_loop(0, _GRP, group, 0)
    pltpu.sync_copy(outv, out_ref.at[pl.ds(row0, _GRP)])


_mf_sc = functools.partial(
    pl.kernel,
    out_type=jax.ShapeDtypeStruct((_NW * _GRP, 16), jnp.float32),
    mesh=plsc.VectorSubcoreMesh(
        core_axis_name="c", subcore_axis_name="s",
        num_cores=_NC, num_subcores=_NS),
    scratch_types=[
        pltpu.VMEM((_GRP, 16), jnp.int32),
        pltpu.VMEM((_GRP, 16), jnp.int32),
        pltpu.VMEM((16, _EDIM, 128), jnp.float32),
        pltpu.VMEM((16, _EDIM, 128), jnp.float32),
        pltpu.VMEM((_GRP, 16), jnp.float32),
        pltpu.SemaphoreType.DMA,
        pltpu.SemaphoreType.DMA,
        pltpu.SemaphoreType.DMA,
        pltpu.SemaphoreType.DMA,
        pltpu.SemaphoreType.DMA,
        pltpu.SemaphoreType.DMA,
        pltpu.SemaphoreType.DMA,
        pltpu.SemaphoreType.DMA,
    ],
)(_mf_body)


def kernel(user, item, U, V):
    u2 = user.astype(jnp.int32).reshape(_NW * _GRP, 16)
    i2 = item.astype(jnp.int32).reshape(_NW * _GRP, 16)
    out = _mf_sc(u2, i2, U.T, V.T)
    return out.reshape(_BATCH)
